# baseline (speedup 1.0000x reference)
import functools

import jax
import jax.numpy as jnp
from jax.experimental import pallas as pl
from jax.experimental.pallas import tpu as pltpu


def _fused_body(x_ref, w1_ref, b1_ref, w2_ref, b2_ref, wh_ref, bh_ref, o_ref,
                *, n_heads):
    x = x_ref[...]
    reg = x[:, -1:]
    h = jnp.maximum(jnp.dot(x.astype(jnp.bfloat16), w1_ref[...],
                            preferred_element_type=jnp.float32) + b1_ref[...], 0.0)
    h = jnp.maximum(jnp.dot(h.astype(jnp.bfloat16), w2_ref[...],
                            preferred_element_type=jnp.float32) + b2_ref[...], 0.0)
    h = h.astype(jnp.bfloat16)
    acc = jnp.zeros(o_ref.shape, jnp.float32)
    for k in range(n_heads):
        lk = jnp.dot(h, wh_ref[k], preferred_element_type=jnp.float32) \
            + bh_ref[k:k + 1, :]
        acc = jnp.where(reg == float(k), lk, acc)
    o_ref[...] = jnp.maximum(acc, 0.0) + jnp.log1p(jnp.exp(-jnp.abs(acc))) + 1.0


@jax.jit
def kernel(x, W1, b1, W2, b2, Wh, bh):
    B, D = x.shape
    H = W1.shape[1]
    K, _, A = Wh.shape
    TB = 1024
    grid = (B // TB,)
    body = functools.partial(_fused_body, n_heads=K)
    return pl.pallas_call(
        body,
        grid=grid,
        in_specs=[
            pl.BlockSpec((TB, D), lambda i: (i, 0)),
            pl.BlockSpec((D, H), lambda i: (0, 0)),
            pl.BlockSpec((1, H), lambda i: (0, 0)),
            pl.BlockSpec((H, H), lambda i: (0, 0)),
            pl.BlockSpec((1, H), lambda i: (0, 0)),
            pl.BlockSpec((K, H, A), lambda i: (0, 0, 0)),
            pl.BlockSpec((K, A), lambda i: (0, 0)),
        ],
        out_specs=pl.BlockSpec((TB, A), lambda i: (i, 0)),
        out_shape=jax.ShapeDtypeStruct((B, A), jnp.float32),
        compiler_params=pltpu.CompilerParams(
            dimension_semantics=("parallel",),
        ),
    )(x, W1.astype(jnp.bfloat16), b1.reshape(1, H), W2.astype(jnp.bfloat16), b2.reshape(1, H), Wh.astype(jnp.bfloat16), bh)


# speedup vs baseline: 1.0437x; 1.0437x over previous
import functools

import jax
import jax.numpy as jnp
from jax.experimental import pallas as pl
from jax.experimental.pallas import tpu as pltpu


def _fused_body(x_ref, w1_ref, b1_ref, w2_ref, b2_ref, wh_ref, bh_ref, o_ref,
                *, n_heads):
    x = x_ref[...]
    reg = x[:, -1:]
    h = jnp.maximum(jnp.dot(x, w1_ref[...],
                            preferred_element_type=jnp.float32) + b1_ref[...], 0.0)
    h = jnp.maximum(jnp.dot(h, w2_ref[...],
                            preferred_element_type=jnp.float32) + b2_ref[...], 0.0)
    acc = jnp.zeros(o_ref.shape, jnp.float32)
    for k in range(n_heads):
        lk = jnp.dot(h, wh_ref[k], preferred_element_type=jnp.float32) \
            + bh_ref[k:k + 1, :]
        acc = jnp.where(reg == float(k), lk, acc)
    o_ref[...] = jnp.maximum(acc, 0.0) + jnp.log1p(jnp.exp(-jnp.abs(acc))) + 1.0


@jax.jit
def kernel(x, W1, b1, W2, b2, Wh, bh):
    B, D = x.shape
    H = W1.shape[1]
    K, _, A = Wh.shape
    TB = 1024
    grid = (B // TB,)
    body = functools.partial(_fused_body, n_heads=K)
    return pl.pallas_call(
        body,
        grid=grid,
        in_specs=[
            pl.BlockSpec((TB, D), lambda i: (i, 0)),
            pl.BlockSpec((D, H), lambda i: (0, 0)),
            pl.BlockSpec((1, H), lambda i: (0, 0)),
            pl.BlockSpec((H, H), lambda i: (0, 0)),
            pl.BlockSpec((1, H), lambda i: (0, 0)),
            pl.BlockSpec((K, H, A), lambda i: (0, 0, 0)),
            pl.BlockSpec((K, A), lambda i: (0, 0)),
        ],
        out_specs=pl.BlockSpec((TB, A), lambda i: (i, 0)),
        out_shape=jax.ShapeDtypeStruct((B, A), jnp.float32),
        compiler_params=pltpu.CompilerParams(
            dimension_semantics=("parallel",),
            vmem_limit_bytes=110 * 1024 * 1024,
        ),
    )(x, W1, b1.reshape(1, H), W2, b2.reshape(1, H), Wh, bh)
